# baseline (speedup 1.0000x reference)
import functools

import jax
import jax.numpy as jnp
from jax import lax
from jax.experimental import pallas as pl
from jax.experimental.pallas import tpu as pltpu
from jax.experimental.pallas import tpu_sc as plsc

N = 10000
P = 10240
DUMMY = N
NC, NS, L = 2, 16, 16
CH = 128
BN = 400


def _rows_per_tile():
    return P // NS


def _deg_body(dst_hbm, zz_hbm, out_hbm, idx_v, ones_v, acc_sh, nch):
    c = lax.axis_index("c")
    s = lax.axis_index("s")
    rpt = _rows_per_tile()
    r0 = s * rpt
    pltpu.sync_copy(zz_hbm.at[pl.ds(r0, rpt)], acc_sh.at[pl.ds(r0, rpt)])
    pltpu.sync_copy(dst_hbm.at[c, s], idx_v)
    ones16 = jnp.ones((16,), jnp.float32)

    def fill(i, carry):
        ones_v[i, :] = ones16
        return carry

    lax.fori_loop(0, CH, fill, 0)
    plsc.subcore_barrier()

    def body(j, carry):
        pltpu.sync_copy(ones_v, acc_sh.at[idx_v.at[j]], add=True)
        return carry

    lax.fori_loop(0, nch, body, 0)
    plsc.subcore_barrier()
    pltpu.sync_copy(acc_sh.at[pl.ds(r0, rpt)], out_hbm.at[c, pl.ds(r0, rpt)])


def _make_deg_kernel(nch):
    return pl.kernel(
        functools.partial(_deg_body, nch=nch),
        out_type=jax.ShapeDtypeStruct((NC, P, 16), jnp.float32),
        mesh=plsc.VectorSubcoreMesh(core_axis_name="c", subcore_axis_name="s"),
        scratch_types=[
            pltpu.VMEM((nch, CH), jnp.int32),
            pltpu.VMEM((CH, 16), jnp.float32),
            pltpu.VMEM_SHARED((P, 16), jnp.float32),
        ],
        compiler_params=pltpu.CompilerParams(use_tc_tiling_on_sc=False),
    )


def _agg_body(table_hbm, src_hbm, dst_hbm, zz_hbm, out_hbm,
              sidx_v, didx_v, gb, acc_sh, sg, ss, nch, feat, nphase, dst_has_core):
    c = lax.axis_index("c")
    s = lax.axis_index("s")
    rpt = _rows_per_tile()
    r0 = s * rpt
    if dst_has_core:
        pltpu.sync_copy(dst_hbm.at[c, s], didx_v)
    else:
        pltpu.sync_copy(dst_hbm.at[s], didx_v)

    for ph in range(nphase):
        pltpu.sync_copy(zz_hbm.at[pl.ds(r0, rpt)], acc_sh.at[pl.ds(r0, rpt)])
        pltpu.sync_copy(src_hbm.at[ph, c, s], sidx_v)
        plsc.subcore_barrier()

        for b in range(3):
            pltpu.async_copy(table_hbm.at[sidx_v.at[b]], gb[b], sg[b])

        def outer(o, carry):
            for b in range(4):
                j = 4 * o + b
                pltpu.make_async_copy(table_hbm.at[sidx_v.at[j]], gb[b], sg[b]).wait()
                pltpu.async_copy(gb[b], acc_sh.at[didx_v.at[j]], ss[b], add=True)
                bb = (b + 3) % 4

                @pl.when(j + 3 < nch)
                def _():
                    @pl.when(j >= 1)
                    def _():
                        jw = jnp.maximum(j - 1, 0)
                        pltpu.make_async_copy(
                            gb[bb], acc_sh.at[didx_v.at[jw]], ss[bb]).wait()

                    pltpu.async_copy(table_hbm.at[sidx_v.at[j + 3]], gb[bb], sg[bb])
            return carry

        lax.fori_loop(0, nch // 4, outer, 0)
        for k in range(4):
            pltpu.make_async_copy(gb[k], acc_sh.at[didx_v.at[nch - 4 + k]], ss[k]).wait()
        plsc.subcore_barrier()
        pltpu.sync_copy(acc_sh.at[pl.ds(r0, rpt)], out_hbm.at[ph, c, pl.ds(r0, rpt)])
        if ph + 1 < nphase:
            plsc.subcore_barrier()


def _make_agg_kernel(nch, feat, nphase, dst_has_core):
    return pl.kernel(
        functools.partial(_agg_body, nch=nch, feat=feat, nphase=nphase,
                          dst_has_core=dst_has_core),
        out_type=jax.ShapeDtypeStruct((nphase, NC, P, feat), jnp.float32),
        mesh=plsc.VectorSubcoreMesh(core_axis_name="c", subcore_axis_name="s"),
        scratch_types=[
            pltpu.VMEM((nch, CH), jnp.int32),
            pltpu.VMEM((nch, CH), jnp.int32),
            [pltpu.VMEM((CH, feat), jnp.float32)] * 4,
            pltpu.VMEM_SHARED((P, feat), jnp.float32),
            [pltpu.SemaphoreType.DMA] * 4,
            [pltpu.SemaphoreType.DMA] * 4,
        ],
        compiler_params=pltpu.CompilerParams(use_tc_tiling_on_sc=False),
    )


def _dinv_block(dga_ref):
    deg = 1.0 + dga_ref[0, :, 0] + dga_ref[1, :, 0]
    return lax.rsqrt(deg)[:, None]


def _tc1_body(x_ref, w1_ref, b1_ref, dga_ref, out_ref):
    dinv = _dinv_block(dga_ref)
    hs = (jnp.dot(x_ref[...], w1_ref[...], preferred_element_type=jnp.float32)
          + b1_ref[...]) * dinv
    out_ref[0] = hs[:, 0:64]
    out_ref[1] = hs[:, 64:128]
    out_ref[2] = hs[:, 128:192]
    out_ref[3] = hs[:, 192:256]


def _tc2_body(agg_ref, hst_ref, dga_ref, w2_ref, b2_ref, out_ref):
    dinv = _dinv_block(dga_ref)
    tot = jnp.concatenate(
        [agg_ref[q] + hst_ref[q] for q in range(4)], axis=1)
    h = jnp.maximum(tot * dinv, 0.0)
    out_ref[...] = (jnp.dot(h, w2_ref[...], preferred_element_type=jnp.float32)
                    + b2_ref[...]) * dinv


def _tc3_body(agg_ref, gs_ref, dga_ref, out_ref):
    dinv = _dinv_block(dga_ref)
    out_ref[...] = (agg_ref[0] + agg_ref[1] + gs_ref[...]) * dinv


def _tc1(x, w1, b1, dga):
    grid = N // BN
    return pl.pallas_call(
        _tc1_body,
        grid=(grid,),
        in_specs=[
            pl.BlockSpec((BN, 128), lambda i: (i, 0)),
            pl.BlockSpec((128, 256), lambda i: (0, 0)),
            pl.BlockSpec((1, 256), lambda i: (0, 0)),
            pl.BlockSpec((2, BN, 16), lambda i: (0, i, 0)),
        ],
        out_specs=pl.BlockSpec((4, BN, 64), lambda i: (0, i, 0)),
        out_shape=jax.ShapeDtypeStruct((4, N, 64), jnp.float32),
    )(x, w1, b1, dga)


def _tc2(agg, hst, dga, w2, b2):
    grid = N // BN
    return pl.pallas_call(
        _tc2_body,
        grid=(grid,),
        in_specs=[
            pl.BlockSpec((4, BN, 64), lambda i: (0, i, 0)),
            pl.BlockSpec((4, BN, 64), lambda i: (0, i, 0)),
            pl.BlockSpec((2, BN, 16), lambda i: (0, i, 0)),
            pl.BlockSpec((256, 64), lambda i: (0, 0)),
            pl.BlockSpec((1, 64), lambda i: (0, 0)),
        ],
        out_specs=pl.BlockSpec((BN, 64), lambda i: (i, 0)),
        out_shape=jax.ShapeDtypeStruct((N, 64), jnp.float32),
    )(agg, hst, dga, w2, b2)


def _tc3(agg, gs, dga):
    grid = N // BN
    return pl.pallas_call(
        _tc3_body,
        grid=(grid,),
        in_specs=[
            pl.BlockSpec((2, BN, 64), lambda i: (0, i, 0)),
            pl.BlockSpec((BN, 64), lambda i: (i, 0)),
            pl.BlockSpec((2, BN, 16), lambda i: (0, i, 0)),
        ],
        out_specs=pl.BlockSpec((BN, 64), lambda i: (i, 0)),
        out_shape=jax.ShapeDtypeStruct((N, 64), jnp.float32),
    )(agg, gs, dga)


def _pad_chunks(a, per, nch, fill):
    g = a.shape[0]
    pad = nch * CH - per
    padv = jnp.full((g, pad), fill, jnp.int32)
    return jnp.concatenate([a, padv], axis=1).reshape(g, nch, CH)


def kernel(x, edge_index, W1, b1, W2, b2):
    src = edge_index[0].astype(jnp.int32)
    dst = edge_index[1].astype(jnp.int32)
    E = src.shape[0]

    e1 = E // NS
    nch1 = 4 * (-(-e1 // (4 * CH)))
    s1 = _pad_chunks(src.reshape(NS, e1), e1, nch1, 0)
    d1 = _pad_chunks(dst.reshape(NS, e1), e1, nch1, DUMMY)
    src1 = jnp.stack([jnp.stack([s1, s1 + N]),
                      jnp.stack([s1 + 2 * N, s1 + 3 * N])])

    e2 = E // (NC * NS)
    nch2 = 4 * (-(-e2 // (4 * CH)))
    s2 = _pad_chunks(src.reshape(NC * NS, e2), e2, nch2, 0).reshape(
        1, NC, NS, nch2, CH)
    d2 = _pad_chunks(dst.reshape(NC * NS, e2), e2, nch2, DUMMY).reshape(
        NC, NS, nch2, CH)

    zz16 = jnp.zeros((P, 16), jnp.float32)
    zz64 = jnp.zeros((P, 64), jnp.float32)
    b1r = b1.reshape(1, 256)
    b2r = b2.reshape(1, 64)

    dga = _make_deg_kernel(nch2)(d2, zz16)

    hst = _tc1(x, W1, b1r, dga)
    table1 = hst.reshape(4 * N, 64)
    agg1 = _make_agg_kernel(nch1, 64, 2, False)(table1, src1, d1, zz64)
    agg1 = agg1.reshape(4, P, 64)[:, :N, :]

    gs = _tc2(agg1, hst, dga, W2, b2r)
    agg2 = _make_agg_kernel(nch2, 64, 1, True)(gs, s2, d2, zz64)
    agg2 = agg2[0, :, :N, :]

    return _tc3(agg2, gs, dga)


# speedup vs baseline: 1.5810x; 1.5810x over previous
import functools

import jax
import jax.numpy as jnp
from jax import lax
from jax.experimental import pallas as pl
from jax.experimental.pallas import tpu as pltpu
from jax.experimental.pallas import tpu_sc as plsc

N = 10000
P = 10240
DUMMY = N
NC, NS, L = 2, 16, 16
CH = 128
BN = 400


def _rows_per_tile():
    return P // NS


def _deg_body(dst_hbm, zz_hbm, out_hbm, idx_v, ones_v, acc_sh, nch):
    c = lax.axis_index("c")
    s = lax.axis_index("s")
    rpt = _rows_per_tile()
    r0 = s * rpt
    pltpu.sync_copy(zz_hbm.at[pl.ds(r0, rpt)], acc_sh.at[pl.ds(r0, rpt)])
    pltpu.sync_copy(dst_hbm.at[c, s], idx_v)
    ones16 = jnp.ones((16,), jnp.float32)

    def fill(i, carry):
        ones_v[i, :] = ones16
        return carry

    lax.fori_loop(0, CH, fill, 0)
    plsc.subcore_barrier()

    def body(j, carry):
        pltpu.sync_copy(ones_v, acc_sh.at[idx_v.at[j]], add=True)
        return carry

    lax.fori_loop(0, nch, body, 0)
    plsc.subcore_barrier()
    pltpu.sync_copy(acc_sh.at[pl.ds(r0, rpt)], out_hbm.at[c, pl.ds(r0, rpt)])


def _make_deg_kernel(nch):
    return pl.kernel(
        functools.partial(_deg_body, nch=nch),
        out_type=jax.ShapeDtypeStruct((NC, P, 16), jnp.float32),
        mesh=plsc.VectorSubcoreMesh(core_axis_name="c", subcore_axis_name="s"),
        scratch_types=[
            pltpu.VMEM((nch, CH), jnp.int32),
            pltpu.VMEM((CH, 16), jnp.float32),
            pltpu.VMEM_SHARED((P, 16), jnp.float32),
        ],
        compiler_params=pltpu.CompilerParams(use_tc_tiling_on_sc=False),
    )


def _agg_body(table_hbm, src_hbm, dst_hbm, zz_hbm, out_hbm,
              sidx_v, didx_v, buf0, buf1, acc_sh, sem0, sem1,
              nch, feat, dst_has_core):
    c = lax.axis_index("c")
    s = lax.axis_index("s")
    rpt = _rows_per_tile()
    r0 = s * rpt
    pltpu.sync_copy(zz_hbm.at[pl.ds(r0, rpt)], acc_sh.at[pl.ds(r0, rpt)])
    pltpu.sync_copy(src_hbm.at[c, s], sidx_v)
    if dst_has_core:
        pltpu.sync_copy(dst_hbm.at[c, s], didx_v)
    else:
        pltpu.sync_copy(dst_hbm.at[s], didx_v)
    plsc.subcore_barrier()

    pltpu.make_async_copy(table_hbm.at[sidx_v.at[0]], buf0, sem0).start()

    def step(j, carry):
        even = (j % 2) == 0

        @pl.when(even)
        def _():
            pltpu.make_async_copy(table_hbm.at[sidx_v.at[j]], buf0, sem0).wait()

            @pl.when(j + 1 < nch)
            def _():
                pltpu.make_async_copy(table_hbm.at[sidx_v.at[j + 1]], buf1, sem1).start()

            pltpu.sync_copy(buf0, acc_sh.at[didx_v.at[j]], add=True)

        @pl.when(jnp.logical_not(even))
        def _():
            pltpu.make_async_copy(table_hbm.at[sidx_v.at[j]], buf1, sem1).wait()

            @pl.when(j + 1 < nch)
            def _():
                pltpu.make_async_copy(table_hbm.at[sidx_v.at[j + 1]], buf0, sem0).start()

            pltpu.sync_copy(buf1, acc_sh.at[didx_v.at[j]], add=True)

        return carry

    lax.fori_loop(0, nch, step, 0)
    plsc.subcore_barrier()
    pltpu.sync_copy(acc_sh.at[pl.ds(r0, rpt)], out_hbm.at[c, pl.ds(r0, rpt)])


def _make_agg_kernel(nch, feat, dst_has_core):
    return pl.kernel(
        functools.partial(_agg_body, nch=nch, feat=feat, dst_has_core=dst_has_core),
        out_type=jax.ShapeDtypeStruct((NC, P, feat), jnp.float32),
        mesh=plsc.VectorSubcoreMesh(core_axis_name="c", subcore_axis_name="s"),
        scratch_types=[
            pltpu.VMEM((nch, CH), jnp.int32),
            pltpu.VMEM((nch, CH), jnp.int32),
            pltpu.VMEM((CH, feat), jnp.float32),
            pltpu.VMEM((CH, feat), jnp.float32),
            pltpu.VMEM_SHARED((P, feat), jnp.float32),
            pltpu.SemaphoreType.DMA,
            pltpu.SemaphoreType.DMA,
        ],
        compiler_params=pltpu.CompilerParams(use_tc_tiling_on_sc=False),
    )


def _dinv_block(dga_ref):
    deg = 1.0 + dga_ref[0, :, 0] + dga_ref[1, :, 0]
    return lax.rsqrt(deg)[:, None]


def _tc0_body(x_ref, dga_ref, out_ref):
    dinv = _dinv_block(dga_ref)
    xs = x_ref[...] * dinv
    pad = jnp.zeros((xs.shape[0], 15), jnp.float32)
    out_ref[0] = jnp.concatenate([xs[:, :64], dinv, pad], axis=1)
    out_ref[1] = jnp.concatenate([xs[:, 64:], dinv, pad], axis=1)


def _tc12_body(agg_ref, x_ref, dga_ref, w1_ref, b1_ref, w2_ref, b2_ref, out_ref):
    dinv = _dinv_block(dga_ref)
    xs = x_ref[...] * dinv
    am = jnp.concatenate([agg_ref[0, :, :64] + xs[:, :64],
                          agg_ref[1, :, :64] + xs[:, 64:]], axis=1)
    sag = agg_ref[0, :, 64:65] + dinv
    h = jnp.maximum(
        (jnp.dot(am, w1_ref[...], preferred_element_type=jnp.float32)
         + sag * b1_ref[...]) * dinv, 0.0)
    out_ref[...] = (jnp.dot(h, w2_ref[...], preferred_element_type=jnp.float32)
                    + b2_ref[...]) * dinv


def _tc3_body(agg_ref, gs_ref, dga_ref, out_ref):
    dinv = _dinv_block(dga_ref)
    out_ref[...] = (agg_ref[0] + agg_ref[1] + gs_ref[...]) * dinv


def _tc0(x, dga):
    grid = N // BN
    return pl.pallas_call(
        _tc0_body,
        grid=(grid,),
        in_specs=[
            pl.BlockSpec((BN, 128), lambda i: (i, 0)),
            pl.BlockSpec((2, BN, 16), lambda i: (0, i, 0)),
        ],
        out_specs=pl.BlockSpec((2, BN, 80), lambda i: (0, i, 0)),
        out_shape=jax.ShapeDtypeStruct((2, N, 80), jnp.float32),
    )(x, dga)


def _tc12(agg, x, dga, w1, b1, w2, b2):
    grid = N // BN
    return pl.pallas_call(
        _tc12_body,
        grid=(grid,),
        in_specs=[
            pl.BlockSpec((2, BN, 80), lambda i: (0, i, 0)),
            pl.BlockSpec((BN, 128), lambda i: (i, 0)),
            pl.BlockSpec((2, BN, 16), lambda i: (0, i, 0)),
            pl.BlockSpec((128, 256), lambda i: (0, 0)),
            pl.BlockSpec((1, 256), lambda i: (0, 0)),
            pl.BlockSpec((256, 64), lambda i: (0, 0)),
            pl.BlockSpec((1, 64), lambda i: (0, 0)),
        ],
        out_specs=pl.BlockSpec((BN, 64), lambda i: (i, 0)),
        out_shape=jax.ShapeDtypeStruct((N, 64), jnp.float32),
    )(agg, x, dga, w1, b1, w2, b2)


def _tc3(agg, gs, dga):
    grid = N // BN
    return pl.pallas_call(
        _tc3_body,
        grid=(grid,),
        in_specs=[
            pl.BlockSpec((2, BN, 64), lambda i: (0, i, 0)),
            pl.BlockSpec((BN, 64), lambda i: (i, 0)),
            pl.BlockSpec((2, BN, 16), lambda i: (0, i, 0)),
        ],
        out_specs=pl.BlockSpec((BN, 64), lambda i: (i, 0)),
        out_shape=jax.ShapeDtypeStruct((N, 64), jnp.float32),
    )(agg, gs, dga)


def _pad_chunks(a, per, nch, fill):
    g = a.shape[0]
    pad = nch * CH - per
    padv = jnp.full((g, pad), fill, jnp.int32)
    return jnp.concatenate([a, padv], axis=1).reshape(g, nch, CH)


def kernel(x, edge_index, W1, b1, W2, b2):
    src = edge_index[0].astype(jnp.int32)
    dst = edge_index[1].astype(jnp.int32)
    E = src.shape[0]

    e1 = E // NS
    nch1 = -(-e1 // CH)
    s1 = _pad_chunks(src.reshape(NS, e1), e1, nch1, 0)
    d1 = _pad_chunks(dst.reshape(NS, e1), e1, nch1, DUMMY)
    src1 = jnp.stack([s1, s1 + N])

    e2 = E // (NC * NS)
    nch2 = -(-e2 // CH)
    s2 = _pad_chunks(src.reshape(NC * NS, e2), e2, nch2, 0).reshape(
        NC, NS, nch2, CH)
    d2 = _pad_chunks(dst.reshape(NC * NS, e2), e2, nch2, DUMMY).reshape(
        NC, NS, nch2, CH)

    zz16 = jnp.zeros((P, 16), jnp.float32)
    zz64 = jnp.zeros((P, 64), jnp.float32)
    zz80 = jnp.zeros((P, 80), jnp.float32)
    b1r = b1.reshape(1, 256)
    b2r = b2.reshape(1, 64)

    dga = _make_deg_kernel(nch2)(d2, zz16)

    xt = _tc0(x, dga)
    table1 = xt.reshape(2 * N, 80)
    aggx = _make_agg_kernel(nch1, 80, False)(table1, src1, d1, zz80)
    aggx = aggx[:, :N, :]

    gs = _tc12(aggx, x, dga, W1, b1r, W2, b2r)

    agg2 = _make_agg_kernel(nch2, 64, True)(gs, s2, d2, zz64)
    agg2 = agg2[:, :N, :]

    return _tc3(agg2, gs, dga)


# speedup vs baseline: 2.0063x; 1.2690x over previous
import functools

import jax
import jax.numpy as jnp
from jax import lax
from jax.experimental import pallas as pl
from jax.experimental.pallas import tpu as pltpu
from jax.experimental.pallas import tpu_sc as plsc

N = 10000
P = 10240
DUMMY = N
NC, NS, L = 2, 16, 16
CH = 128
BN = 400


def _rows_per_tile():
    return P // NS


def _deg_body(dst_hbm, zz_hbm, out_hbm, idx_v, ones_v, acc_sh, nch):
    c = lax.axis_index("c")
    s = lax.axis_index("s")
    rpt = _rows_per_tile()
    r0 = s * rpt
    pltpu.sync_copy(zz_hbm.at[pl.ds(r0, rpt)], acc_sh.at[pl.ds(r0, rpt)])
    pltpu.sync_copy(dst_hbm.at[c, s], idx_v)
    ones16 = jnp.ones((16,), jnp.float32)

    def fill(i, carry):
        ones_v[i, :] = ones16
        return carry

    lax.fori_loop(0, CH, fill, 0)
    plsc.subcore_barrier()

    def body(j, carry):
        pltpu.sync_copy(ones_v, acc_sh.at[idx_v.at[j]], add=True)
        return carry

    lax.fori_loop(0, nch, body, 0)
    plsc.subcore_barrier()
    pltpu.sync_copy(acc_sh.at[pl.ds(r0, rpt)], out_hbm.at[c, pl.ds(r0, rpt)])


def _make_deg_kernel(nch):
    return pl.kernel(
        functools.partial(_deg_body, nch=nch),
        out_type=jax.ShapeDtypeStruct((NC, P, 16), jnp.float32),
        mesh=plsc.VectorSubcoreMesh(core_axis_name="c", subcore_axis_name="s"),
        scratch_types=[
            pltpu.VMEM((nch, CH), jnp.int32),
            pltpu.VMEM((CH, 16), jnp.float32),
            pltpu.VMEM_SHARED((P, 16), jnp.float32),
        ],
        compiler_params=pltpu.CompilerParams(use_tc_tiling_on_sc=False),
    )


def _agg_body(table_hbm, src_hbm, dst_hbm, zz_hbm, out_hbm,
              sidx_v, didx_v, bufs, acc_sh, sems,
              nch, feat, nb, dst_has_core):
    c = lax.axis_index("c")
    s = lax.axis_index("s")
    rpt = _rows_per_tile()
    r0 = s * rpt
    pltpu.sync_copy(zz_hbm.at[pl.ds(r0, rpt)], acc_sh.at[pl.ds(r0, rpt)])
    pltpu.sync_copy(src_hbm.at[c, s], sidx_v)
    if dst_has_core:
        pltpu.sync_copy(dst_hbm.at[c, s], didx_v)
    else:
        pltpu.sync_copy(dst_hbm.at[s], didx_v)
    plsc.subcore_barrier()

    for t in range(nb - 1):
        pltpu.async_copy(table_hbm.at[sidx_v.at[t]], bufs[t], sems[t])

    def step(j, carry):
        m = j % nb
        for b in range(nb):
            @pl.when(m == b)
            def _(b=b):
                pltpu.make_async_copy(
                    table_hbm.at[sidx_v.at[j]], bufs[b], sems[b]).wait()

                @pl.when(j + nb - 1 < nch)
                def _():
                    bn = (b + nb - 1) % nb
                    pltpu.async_copy(
                        table_hbm.at[sidx_v.at[j + nb - 1]], bufs[bn], sems[bn])

                pltpu.sync_copy(bufs[b], acc_sh.at[didx_v.at[j]], add=True)
        return carry

    lax.fori_loop(0, nch, step, 0)
    plsc.subcore_barrier()
    pltpu.sync_copy(acc_sh.at[pl.ds(r0, rpt)], out_hbm.at[c, pl.ds(r0, rpt)])


def _make_agg_kernel(nch, feat, nb, dst_has_core):
    return pl.kernel(
        functools.partial(_agg_body, nch=nch, feat=feat, nb=nb,
                          dst_has_core=dst_has_core),
        out_type=jax.ShapeDtypeStruct((NC, P, feat), jnp.float32),
        mesh=plsc.VectorSubcoreMesh(core_axis_name="c", subcore_axis_name="s"),
        scratch_types=[
            pltpu.VMEM((nch, CH), jnp.int32),
            pltpu.VMEM((nch, CH), jnp.int32),
            [pltpu.VMEM((CH, feat), jnp.float32)] * nb,
            pltpu.VMEM_SHARED((P, feat), jnp.float32),
            [pltpu.SemaphoreType.DMA] * nb,
        ],
        compiler_params=pltpu.CompilerParams(use_tc_tiling_on_sc=False),
    )


def _dinv_block(dga_ref):
    deg = 1.0 + dga_ref[0, :, 0] + dga_ref[1, :, 0]
    return lax.rsqrt(deg)[:, None]


def _tc0_body(x_ref, dga_ref, out_ref):
    dinv = _dinv_block(dga_ref)
    xs = x_ref[...] * dinv
    pad = jnp.zeros((xs.shape[0], 15), jnp.float32)
    out_ref[0] = jnp.concatenate([xs[:, :64], dinv, pad], axis=1)
    out_ref[1] = jnp.concatenate([xs[:, 64:], dinv, pad], axis=1)


def _tc12_body(agg_ref, x_ref, dga_ref, w1_ref, b1_ref, w2_ref, b2_ref, out_ref):
    dinv = _dinv_block(dga_ref)
    xs = x_ref[...] * dinv
    am = jnp.concatenate([agg_ref[0, :, :64] + xs[:, :64],
                          agg_ref[1, :, :64] + xs[:, 64:]], axis=1)
    sag = agg_ref[0, :, 64:65] + dinv
    h = jnp.maximum(
        (jnp.dot(am, w1_ref[...], preferred_element_type=jnp.float32)
         + sag * b1_ref[...]) * dinv, 0.0)
    out_ref[...] = (jnp.dot(h, w2_ref[...], preferred_element_type=jnp.float32)
                    + b2_ref[...]) * dinv


def _tc3_body(agg_ref, gs_ref, dga_ref, out_ref):
    dinv = _dinv_block(dga_ref)
    out_ref[...] = (agg_ref[0] + agg_ref[1] + gs_ref[...]) * dinv


def _tc0(x, dga):
    grid = N // BN
    return pl.pallas_call(
        _tc0_body,
        grid=(grid,),
        in_specs=[
            pl.BlockSpec((BN, 128), lambda i: (i, 0)),
            pl.BlockSpec((2, BN, 16), lambda i: (0, i, 0)),
        ],
        out_specs=pl.BlockSpec((2, BN, 80), lambda i: (0, i, 0)),
        out_shape=jax.ShapeDtypeStruct((2, N, 80), jnp.float32),
    )(x, dga)


def _tc12(agg, x, dga, w1, b1, w2, b2):
    grid = N // BN
    return pl.pallas_call(
        _tc12_body,
        grid=(grid,),
        in_specs=[
            pl.BlockSpec((2, BN, 80), lambda i: (0, i, 0)),
            pl.BlockSpec((BN, 128), lambda i: (i, 0)),
            pl.BlockSpec((2, BN, 16), lambda i: (0, i, 0)),
            pl.BlockSpec((128, 256), lambda i: (0, 0)),
            pl.BlockSpec((1, 256), lambda i: (0, 0)),
            pl.BlockSpec((256, 64), lambda i: (0, 0)),
            pl.BlockSpec((1, 64), lambda i: (0, 0)),
        ],
        out_specs=pl.BlockSpec((BN, 64), lambda i: (i, 0)),
        out_shape=jax.ShapeDtypeStruct((N, 64), jnp.float32),
    )(agg, x, dga, w1, b1, w2, b2)


def _tc3(agg, gs, dga):
    grid = N // BN
    return pl.pallas_call(
        _tc3_body,
        grid=(grid,),
        in_specs=[
            pl.BlockSpec((2, BN, 64), lambda i: (0, i, 0)),
            pl.BlockSpec((BN, 64), lambda i: (i, 0)),
            pl.BlockSpec((2, BN, 16), lambda i: (0, i, 0)),
        ],
        out_specs=pl.BlockSpec((BN, 64), lambda i: (i, 0)),
        out_shape=jax.ShapeDtypeStruct((N, 64), jnp.float32),
    )(agg, gs, dga)


def _pad_chunks(a, per, nch, fill):
    g = a.shape[0]
    pad = nch * CH - per
    padv = jnp.full((g, pad), fill, jnp.int32)
    return jnp.concatenate([a, padv], axis=1).reshape(g, nch, CH)


def kernel(x, edge_index, W1, b1, W2, b2):
    src = edge_index[0].astype(jnp.int32)
    dst = edge_index[1].astype(jnp.int32)
    E = src.shape[0]

    e1 = E // NS
    nch1 = -(-e1 // CH)
    s1 = _pad_chunks(src.reshape(NS, e1), e1, nch1, 0)
    d1 = _pad_chunks(dst.reshape(NS, e1), e1, nch1, DUMMY)
    src1 = jnp.stack([s1, s1 + N])

    e2 = E // (NC * NS)
    nch2 = -(-e2 // CH)
    s2 = _pad_chunks(src.reshape(NC * NS, e2), e2, nch2, 0).reshape(
        NC, NS, nch2, CH)
    d2 = _pad_chunks(dst.reshape(NC * NS, e2), e2, nch2, DUMMY).reshape(
        NC, NS, nch2, CH)

    zz16 = jnp.zeros((P, 16), jnp.float32)
    zz64 = jnp.zeros((P, 64), jnp.float32)
    zz80 = jnp.zeros((P, 80), jnp.float32)
    b1r = b1.reshape(1, 256)
    b2r = b2.reshape(1, 64)

    dga = _make_deg_kernel(nch2)(d2, zz16)

    xt = _tc0(x, dga)
    table1 = xt.reshape(2 * N, 80)
    aggx = _make_agg_kernel(nch1, 80, 3, False)(table1, src1, d1, zz80)

    gs = _tc12(aggx, x, dga, W1, b1r, W2, b2r)

    agg2 = _make_agg_kernel(nch2, 64, 6, True)(gs, s2, d2, zz64)

    return _tc3(agg2, gs, dga)
